# Initial kernel scaffold; baseline (speedup 1.0000x reference)
import jax
import jax.numpy as jnp
from jax.experimental import pallas as pl


def kernel(x, y, inv_softplus_step_values):
    raise NotImplementedError("write your pallas kernel here")


# speedup vs baseline: 1.7468x; 1.7468x over previous
import functools

import jax
import jax.numpy as jnp
from jax import lax
from jax.experimental import pallas as pl
from jax.experimental.pallas import tpu as pltpu
from jax.experimental.pallas import tpu_sc as plsc

_INPUT_RANGE = 4.0
_NB = 1024
_DX = 2.0 * _INPUT_RANGE / (_NB - 1)
_INV_DX = 1.0 / _DX

_N_TOTAL = 16384 * 100
_NW = 32
_PER_W = _N_TOTAL // _NW
_CH = 512
_NSUB = _CH // 128
_NCHUNK = _PER_W // _CH


def _cint_body(w_ref, out_ref):
    w = w_ref[...]
    sp = jnp.maximum(w, 0.0) + jnp.log1p(jnp.exp(-jnp.abs(w)))
    cell = sp * (_DX * _DX)
    row_i = lax.broadcasted_iota(jnp.int32, (_NB, _NB), 0)
    col_i = lax.broadcasted_iota(jnp.int32, (_NB, _NB), 1)
    tri = (row_i <= col_i).astype(jnp.float32)
    cs1 = lax.dot(cell, tri, precision=lax.Precision.HIGHEST)
    cs2 = lax.dot_general(tri, cs1, (((0,), (0,)), ((), ())),
                          precision=lax.Precision.HIGHEST)
    a = cs2[0:1, 0:1]
    b = cs2[_NB - 1:_NB, _NB - 1:_NB]
    scale = (2.0 * _INPUT_RANGE) / (b - a)
    out_ref[...] = (cs2 - a) * scale - _INPUT_RANGE


def _table_tc(w):
    return pl.pallas_call(
        _cint_body,
        out_shape=jax.ShapeDtypeStruct((_NB, _NB), jnp.float32),
    )(w)


@functools.lru_cache(maxsize=1)
def _make_sc_gather():
  mesh = plsc.VectorSubcoreMesh(core_axis_name="c", subcore_axis_name="s")

  @functools.partial(
    pl.kernel,
    mesh=mesh,
    out_type=jax.ShapeDtypeStruct((_N_TOTAL,), jnp.float32),
    scratch_types=[
        pltpu.VMEM_SHARED((_NB * _NB,), jnp.float32),
        pltpu.VMEM((_CH,), jnp.float32),
        pltpu.VMEM((_CH,), jnp.float32),
        pltpu.VMEM((_NSUB, 128), jnp.int32),
        pltpu.VMEM((_NSUB, 128), jnp.int32),
        pltpu.VMEM((_NSUB, 128), jnp.int32),
        pltpu.VMEM((_NSUB, 128), jnp.int32),
        pltpu.VMEM((_CH,), jnp.float32),
        pltpu.VMEM((_CH,), jnp.float32),
        pltpu.VMEM((_NSUB, 128), jnp.float32),
        pltpu.VMEM((_NSUB, 128), jnp.float32),
        pltpu.VMEM((_NSUB, 128), jnp.float32),
        pltpu.VMEM((_NSUB, 128), jnp.float32),
        pltpu.VMEM((_CH,), jnp.float32),
        pltpu.SemaphoreType.DMA,
    ],
  )
  def _sc_gather(x_hbm, y_hbm, cint_hbm, out_hbm, table_sh,
               xv, yv, i00, i01, i10, i11, fxv, fyv,
               g00, g01, g10, g11, ov, sem):
    s_id = lax.axis_index("s")
    wid = s_id * 2 + lax.axis_index("c")

    seg = (_NB * _NB) // 16
    pltpu.sync_copy(cint_hbm.at[pl.ds(s_id * seg, seg)],
                    table_sh.at[pl.ds(s_id * seg, seg)])
    plsc.subcore_barrier()

    def chunk_body(t, carry):
        base = wid * _PER_W + t * _CH
        pltpu.sync_copy(x_hbm.at[pl.ds(base, _CH)], xv)
        pltpu.sync_copy(y_hbm.at[pl.ds(base, _CH)], yv)
        for i in range(_CH // 16):
            p = i * 16
            sub, off = p // 128, p % 128
            u = xv[pl.ds(p, 16)] * _INV_DX + (_INPUT_RANGE * _INV_DX)
            v = yv[pl.ds(p, 16)] * _INV_DX + (_INPUT_RANGE * _INV_DX)
            cx = jnp.clip(u.astype(jnp.int32), 0, _NB - 2)
            cy = jnp.clip(v.astype(jnp.int32), 0, _NB - 2)
            fxv[pl.ds(p, 16)] = u - cx.astype(jnp.float32)
            fyv[pl.ds(p, 16)] = v - cy.astype(jnp.float32)
            base_idx = cy * _NB + cx
            i00[sub, pl.ds(off, 16)] = base_idx
            i01[sub, pl.ds(off, 16)] = base_idx + 1
            i10[sub, pl.ds(off, 16)] = base_idx + _NB
            i11[sub, pl.ds(off, 16)] = base_idx + (_NB + 1)
        copies = []
        for j in range(_NSUB):
            copies.append(pltpu.async_copy(table_sh.at[i00.at[j]], g00.at[j], sem))
            copies.append(pltpu.async_copy(table_sh.at[i01.at[j]], g01.at[j], sem))
            copies.append(pltpu.async_copy(table_sh.at[i10.at[j]], g10.at[j], sem))
            copies.append(pltpu.async_copy(table_sh.at[i11.at[j]], g11.at[j], sem))
        for c in copies:
            c.wait()
        for i in range(_CH // 16):
            p = i * 16
            sub, off = p // 128, p % 128
            z00 = g00[sub, pl.ds(off, 16)]
            z01 = g01[sub, pl.ds(off, 16)]
            z10 = g10[sub, pl.ds(off, 16)]
            z11 = g11[sub, pl.ds(off, 16)]
            fx = fxv[pl.ds(p, 16)]
            fy = fyv[pl.ds(p, 16)]
            top = z00 + fx * (z01 - z00)
            bot = z10 + fx * (z11 - z10)
            ov[pl.ds(p, 16)] = top + fy * (bot - top)
        pltpu.sync_copy(ov, out_hbm.at[pl.ds(base, _CH)])
        return carry

    lax.fori_loop(0, _NCHUNK, chunk_body, 0)

  return _sc_gather


def kernel(x, y, inv_softplus_step_values):
    cint = _table_tc(inv_softplus_step_values)
    out = _make_sc_gather()(x.reshape(-1), y.reshape(-1), cint.reshape(-1))
    return out.reshape(x.shape)


# speedup vs baseline: 1.9566x; 1.1201x over previous
import functools

import jax
import jax.numpy as jnp
from jax import lax
from jax.experimental import pallas as pl
from jax.experimental.pallas import tpu as pltpu
from jax.experimental.pallas import tpu_sc as plsc

_INPUT_RANGE = 4.0
_NB = 1024
_DX = 2.0 * _INPUT_RANGE / (_NB - 1)
_INV_DX = 1.0 / _DX

_N_TOTAL = 16384 * 100
_NW = 32
_PER_W = _N_TOTAL // _NW
_CH = 512
_NSUB = _CH // 128
_NCHUNK = _PER_W // _CH


def _cint_body(w_ref, out_ref):
    w = w_ref[...]
    sp = jnp.maximum(w, 0.0) + jnp.log1p(jnp.exp(-jnp.abs(w)))
    cell = sp * (_DX * _DX)
    row_i = lax.broadcasted_iota(jnp.int32, (_NB, _NB), 0)
    col_i = lax.broadcasted_iota(jnp.int32, (_NB, _NB), 1)
    tri = (row_i <= col_i).astype(jnp.float32)
    cs1 = lax.dot(cell, tri, precision=lax.Precision.HIGHEST)
    cs2 = lax.dot_general(tri, cs1, (((0,), (0,)), ((), ())),
                          precision=lax.Precision.HIGHEST)
    a = cs2[0:1, 0:1]
    b = cs2[_NB - 1:_NB, _NB - 1:_NB]
    scale = (2.0 * _INPUT_RANGE) / (b - a)
    cint = (cs2 - a) * scale - _INPUT_RANGE
    cint_r = jnp.concatenate([cint[:, 1:], cint[:, _NB - 1:_NB]], axis=1)
    lo = lax.bitcast_convert_type(cint.astype(jnp.bfloat16),
                                  jnp.uint16).astype(jnp.uint32)
    hi = lax.bitcast_convert_type(cint_r.astype(jnp.bfloat16),
                                  jnp.uint16).astype(jnp.uint32)
    out_ref[...] = (lo | (hi << 16)).astype(jnp.int32)


def _table_tc(w):
    return pl.pallas_call(
        _cint_body,
        out_shape=jax.ShapeDtypeStruct((_NB, _NB), jnp.int32),
    )(w)


@functools.lru_cache(maxsize=1)
def _make_sc_gather():
  mesh = plsc.VectorSubcoreMesh(core_axis_name="c", subcore_axis_name="s")

  @functools.partial(
    pl.kernel,
    mesh=mesh,
    out_type=jax.ShapeDtypeStruct((_N_TOTAL,), jnp.float32),
    scratch_types=[
        pltpu.VMEM_SHARED((_NB * _NB,), jnp.int32),
        pltpu.VMEM((_CH,), jnp.float32),
        pltpu.VMEM((_CH,), jnp.float32),
        pltpu.VMEM((_NSUB, 128), jnp.int32),
        pltpu.VMEM((_NSUB, 128), jnp.int32),
        pltpu.VMEM((_CH,), jnp.float32),
        pltpu.VMEM((_CH,), jnp.float32),
        pltpu.VMEM((_NSUB, 128), jnp.int32),
        pltpu.VMEM((_NSUB, 128), jnp.int32),
        pltpu.VMEM((_CH,), jnp.float32),
        pltpu.SemaphoreType.DMA,
    ],
  )
  def _sc_gather(x_hbm, y_hbm, table_hbm, out_hbm, table_sh,
                 xv, yv, i00, i10, fxv, fyv, g0, g1, ov, sem):
    s_id = lax.axis_index("s")
    wid = s_id * 2 + lax.axis_index("c")

    seg = (_NB * _NB) // 16
    pltpu.sync_copy(table_hbm.at[pl.ds(s_id * seg, seg)],
                    table_sh.at[pl.ds(s_id * seg, seg)])
    plsc.subcore_barrier()

    def chunk_body(t, carry):
        base = wid * _PER_W + t * _CH
        pltpu.sync_copy(x_hbm.at[pl.ds(base, _CH)], xv)
        pltpu.sync_copy(y_hbm.at[pl.ds(base, _CH)], yv)
        for i in range(_CH // 16):
            p = i * 16
            sub, off = p // 128, p % 128
            u = xv[pl.ds(p, 16)] * _INV_DX + (_INPUT_RANGE * _INV_DX)
            v = yv[pl.ds(p, 16)] * _INV_DX + (_INPUT_RANGE * _INV_DX)
            cx = jnp.clip(u.astype(jnp.int32), 0, _NB - 2)
            cy = jnp.clip(v.astype(jnp.int32), 0, _NB - 2)
            fxv[pl.ds(p, 16)] = u - cx.astype(jnp.float32)
            fyv[pl.ds(p, 16)] = v - cy.astype(jnp.float32)
            base_idx = cy * _NB + cx
            i00[sub, pl.ds(off, 16)] = base_idx
            i10[sub, pl.ds(off, 16)] = base_idx + _NB
        copies = []
        for j in range(_NSUB):
            copies.append(pltpu.async_copy(table_sh.at[i00.at[j]], g0.at[j], sem))
            copies.append(pltpu.async_copy(table_sh.at[i10.at[j]], g1.at[j], sem))
        for c in copies:
            c.wait()
        for i in range(_CH // 16):
            p = i * 16
            sub, off = p // 128, p % 128
            w0 = g0[sub, pl.ds(off, 16)]
            w1 = g1[sub, pl.ds(off, 16)]
            z00 = lax.bitcast_convert_type(w0 << 16, jnp.float32)
            z01 = lax.bitcast_convert_type(w0 & jnp.int32(-65536), jnp.float32)
            z10 = lax.bitcast_convert_type(w1 << 16, jnp.float32)
            z11 = lax.bitcast_convert_type(w1 & jnp.int32(-65536), jnp.float32)
            fx = fxv[pl.ds(p, 16)]
            fy = fyv[pl.ds(p, 16)]
            top = z00 + fx * (z01 - z00)
            bot = z10 + fx * (z11 - z10)
            ov[pl.ds(p, 16)] = top + fy * (bot - top)
        pltpu.sync_copy(ov, out_hbm.at[pl.ds(base, _CH)])
        return carry

    lax.fori_loop(0, _NCHUNK, chunk_body, 0)

  return _sc_gather


def kernel(x, y, inv_softplus_step_values):
    table = _table_tc(inv_softplus_step_values)
    out = _make_sc_gather()(x.reshape(-1), y.reshape(-1), table.reshape(-1))
    return out.reshape(x.shape)


# speedup vs baseline: 3.6405x; 1.8607x over previous
import functools

import jax
import jax.numpy as jnp
from jax import lax
from jax.experimental import pallas as pl
from jax.experimental.pallas import tpu as pltpu
from jax.experimental.pallas import tpu_sc as plsc

_INPUT_RANGE = 4.0
_NB = 1024
_DX = 2.0 * _INPUT_RANGE / (_NB - 1)
_INV_DX = 1.0 / _DX

_N_TOTAL = 16384 * 100
_NW = 32
_PER_W = _N_TOTAL // _NW
_CH = 512
_NSUB = _CH // 128
_NCHUNK = _PER_W // _CH


def _cint_body(w_ref, out_ref):
    w = w_ref[...]
    sp = jnp.maximum(w, 0.0) + jnp.log1p(jnp.exp(-jnp.abs(w)))
    cell = sp * (_DX * _DX)
    row_i = lax.broadcasted_iota(jnp.int32, (_NB, _NB), 0)
    col_i = lax.broadcasted_iota(jnp.int32, (_NB, _NB), 1)
    tri = (row_i <= col_i).astype(jnp.float32)
    cs1 = lax.dot(cell, tri, precision=lax.Precision.HIGHEST)
    cs2 = lax.dot_general(tri, cs1, (((0,), (0,)), ((), ())),
                          precision=lax.Precision.HIGHEST)
    a = cs2[0:1, 0:1]
    b = cs2[_NB - 1:_NB, _NB - 1:_NB]
    scale = (2.0 * _INPUT_RANGE) / (b - a)
    cint = (cs2 - a) * scale - _INPUT_RANGE
    cint_r = jnp.concatenate([cint[:, 1:], cint[:, _NB - 1:_NB]], axis=1)
    lo = lax.bitcast_convert_type(cint.astype(jnp.bfloat16),
                                  jnp.uint16).astype(jnp.uint32)
    hi = lax.bitcast_convert_type(cint_r.astype(jnp.bfloat16),
                                  jnp.uint16).astype(jnp.uint32)
    out_ref[...] = (lo | (hi << 16)).astype(jnp.int32)


def _table_tc(w):
    return pl.pallas_call(
        _cint_body,
        out_shape=jax.ShapeDtypeStruct((_NB, _NB), jnp.int32),
    )(w)


@functools.lru_cache(maxsize=1)
def _make_sc_gather():
  mesh = plsc.VectorSubcoreMesh(core_axis_name="c", subcore_axis_name="s")

  buf = lambda dt: pltpu.VMEM((_CH,), dt)
  idxbuf = lambda: pltpu.VMEM((_NSUB, 128), jnp.int32)

  @functools.partial(
    pl.kernel,
    mesh=mesh,
    out_type=jax.ShapeDtypeStruct((_N_TOTAL,), jnp.float32),
    scratch_types=[
        pltpu.VMEM_SHARED((_NB * _NB,), jnp.int32),
        [buf(jnp.float32), buf(jnp.float32)],
        [buf(jnp.float32), buf(jnp.float32)],
        [idxbuf(), idxbuf()],
        [idxbuf(), idxbuf()],
        [buf(jnp.float32), buf(jnp.float32)],
        [buf(jnp.float32), buf(jnp.float32)],
        [idxbuf(), idxbuf()],
        [idxbuf(), idxbuf()],
        [buf(jnp.float32), buf(jnp.float32)],
        [pltpu.SemaphoreType.DMA, pltpu.SemaphoreType.DMA],
        [pltpu.SemaphoreType.DMA, pltpu.SemaphoreType.DMA],
        [pltpu.SemaphoreType.DMA, pltpu.SemaphoreType.DMA],
    ],
  )
  def _sc_gather(x_hbm, y_hbm, table_hbm, out_hbm, table_sh,
                 xv, yv, i00, i10, fxv, fyv, g0, g1, ov,
                 sem_in, sem_g, sem_out):
    s_id = lax.axis_index("s")
    wid = s_id * 2 + lax.axis_index("c")
    wbase = wid * _PER_W

    seg = (_NB * _NB) // 16
    pltpu.sync_copy(table_hbm.at[pl.ds(s_id * seg, seg)],
                    table_sh.at[pl.ds(s_id * seg, seg)])
    plsc.subcore_barrier()

    def fire_in(t, b):
        base = wbase + t * _CH
        pltpu.async_copy(x_hbm.at[pl.ds(base, _CH)], xv[b], sem_in[b])
        pltpu.async_copy(y_hbm.at[pl.ds(base, _CH)], yv[b], sem_in[b])

    def wait_in(b):
        pltpu.make_async_copy(x_hbm.at[pl.ds(0, _CH)], xv[b], sem_in[b]).wait()
        pltpu.make_async_copy(y_hbm.at[pl.ds(0, _CH)], yv[b], sem_in[b]).wait()

    def compute_idx(b):
        for i in range(_CH // 16):
            p = i * 16
            sub, off = p // 128, p % 128
            u = xv[b][pl.ds(p, 16)] * _INV_DX + (_INPUT_RANGE * _INV_DX)
            v = yv[b][pl.ds(p, 16)] * _INV_DX + (_INPUT_RANGE * _INV_DX)
            cx = jnp.clip(u.astype(jnp.int32), 0, _NB - 2)
            cy = jnp.clip(v.astype(jnp.int32), 0, _NB - 2)
            fxv[b][pl.ds(p, 16)] = u - cx.astype(jnp.float32)
            fyv[b][pl.ds(p, 16)] = v - cy.astype(jnp.float32)
            base_idx = cy * _NB + cx
            i00[b][sub, pl.ds(off, 16)] = base_idx
            i10[b][sub, pl.ds(off, 16)] = base_idx + _NB

    def fire_gathers(b):
        for j in range(_NSUB):
            pltpu.async_copy(table_sh.at[i00[b].at[j]], g0[b].at[j], sem_g[b])
            pltpu.async_copy(table_sh.at[i10[b].at[j]], g1[b].at[j], sem_g[b])

    def wait_gathers(b):
        for j in range(_NSUB):
            pltpu.make_async_copy(
                table_hbm.at[pl.ds(0, 128)], g0[b].at[j], sem_g[b]).wait()
            pltpu.make_async_copy(
                table_hbm.at[pl.ds(0, 128)], g1[b].at[j], sem_g[b]).wait()

    def combine(b):
        for i in range(_CH // 16):
            p = i * 16
            sub, off = p // 128, p % 128
            w0 = g0[b][sub, pl.ds(off, 16)]
            w1 = g1[b][sub, pl.ds(off, 16)]
            z00 = lax.bitcast_convert_type(w0 << 16, jnp.float32)
            z01 = lax.bitcast_convert_type(w0 & jnp.int32(-65536), jnp.float32)
            z10 = lax.bitcast_convert_type(w1 << 16, jnp.float32)
            z11 = lax.bitcast_convert_type(w1 & jnp.int32(-65536), jnp.float32)
            fx = fxv[b][pl.ds(p, 16)]
            fy = fyv[b][pl.ds(p, 16)]
            top = z00 + fx * (z01 - z00)
            bot = z10 + fx * (z11 - z10)
            ov[b][pl.ds(p, 16)] = top + fy * (bot - top)

    def fire_out(t, b):
        base = wbase + t * _CH
        pltpu.async_copy(ov[b], out_hbm.at[pl.ds(base, _CH)], sem_out[b])

    def wait_out(b):
        pltpu.make_async_copy(
            x_hbm.at[pl.ds(0, _CH)], ov[b], sem_out[b]).wait()

    fire_in(0, 0)
    fire_in(1, 1)

    def body(k, carry):
        t = 2 * k
        wait_in(0)
        compute_idx(0)

        @pl.when(t + 2 < _NCHUNK)
        def _():
            fire_in(t + 2, 0)
        fire_gathers(0)

        @pl.when(k > 0)
        def _():
            wait_gathers(1)

            @pl.when(k > 1)
            def _():
                wait_out(1)
            combine(1)
            fire_out(t - 1, 1)

        wait_in(1)
        compute_idx(1)

        @pl.when(t + 3 < _NCHUNK)
        def _():
            fire_in(t + 3, 1)
        fire_gathers(1)

        wait_gathers(0)

        @pl.when(k > 0)
        def _():
            wait_out(0)
        combine(0)
        fire_out(t, 0)
        return carry

    lax.fori_loop(0, _NCHUNK // 2, body, 0)

    wait_gathers(1)
    wait_out(1)
    combine(1)
    fire_out(_NCHUNK - 1, 1)
    wait_out(0)
    wait_out(1)

  return _sc_gather


def kernel(x, y, inv_softplus_step_values):
    table = _table_tc(inv_softplus_step_values)
    out = _make_sc_gather()(x.reshape(-1), y.reshape(-1), table.reshape(-1))
    return out.reshape(x.shape)


# speedup vs baseline: 4.2745x; 1.1741x over previous
import functools

import jax
import jax.numpy as jnp
from jax import lax
from jax.experimental import pallas as pl
from jax.experimental.pallas import tpu as pltpu
from jax.experimental.pallas import tpu_sc as plsc

_INPUT_RANGE = 4.0
_NB = 1024
_DX = 2.0 * _INPUT_RANGE / (_NB - 1)
_INV_DX = 1.0 / _DX

_ROWS, _COLS = 16384, 100
_NW = 32
_ROWS_W = _ROWS // _NW
_RCH = 8
_NCHUNK = _ROWS_W // _RCH
_VPR = 7
_NVS = _RCH * _VPR
_NSUB = (_NVS * 16) // 128


def _cint_body(w_ref, out_ref):
    w = w_ref[...]
    sp = jnp.maximum(w, 0.0) + jnp.log1p(jnp.exp(-jnp.abs(w)))
    cell = sp * (_DX * _DX)
    row_i = lax.broadcasted_iota(jnp.int32, (_NB, _NB), 0)
    col_i = lax.broadcasted_iota(jnp.int32, (_NB, _NB), 1)
    tri = (row_i <= col_i).astype(jnp.float32)
    cs1 = lax.dot(cell, tri)
    cs2 = lax.dot_general(tri, cs1, (((0,), (0,)), ((), ())))
    a = cs2[0:1, 0:1]
    b = cs2[_NB - 1:_NB, _NB - 1:_NB]
    scale = (2.0 * _INPUT_RANGE) / (b - a)
    cint = (cs2 - a) * scale - _INPUT_RANGE
    cint_r = jnp.concatenate([cint[:, 1:], cint[:, _NB - 1:_NB]], axis=1)
    lo = lax.bitcast_convert_type(cint.astype(jnp.bfloat16),
                                  jnp.uint16).astype(jnp.uint32)
    hi = lax.bitcast_convert_type(cint_r.astype(jnp.bfloat16),
                                  jnp.uint16).astype(jnp.uint32)
    out_ref[...] = (lo | (hi << 16)).astype(jnp.int32)


def _table_tc(w):
    return pl.pallas_call(
        _cint_body,
        out_shape=jax.ShapeDtypeStruct((_NB, _NB), jnp.int32),
    )(w)


def _cstart(j):
    return 16 * j if j < _VPR - 1 else _COLS - 16


@functools.lru_cache(maxsize=1)
def _make_sc_gather():
  mesh = plsc.VectorSubcoreMesh(core_axis_name="c", subcore_axis_name="s")

  io2d = lambda: pltpu.VMEM((_RCH, _COLS), jnp.float32)
  flatbuf = lambda: pltpu.VMEM((_NVS * 16,), jnp.float32)
  idxbuf = lambda: pltpu.VMEM((_NSUB, 128), jnp.int32)
  gbuf = lambda: pltpu.VMEM((_NSUB, 128), jnp.int32)

  @functools.partial(
    pl.kernel,
    mesh=mesh,
    out_type=jax.ShapeDtypeStruct((_ROWS, _COLS), jnp.float32),
    scratch_types=[
        pltpu.VMEM_SHARED((_NB * _NB,), jnp.int32),
        [io2d(), io2d()],
        [io2d(), io2d()],
        [idxbuf(), idxbuf()],
        [idxbuf(), idxbuf()],
        [flatbuf(), flatbuf()],
        [flatbuf(), flatbuf()],
        [gbuf(), gbuf()],
        [gbuf(), gbuf()],
        [io2d(), io2d()],
        [pltpu.SemaphoreType.DMA, pltpu.SemaphoreType.DMA],
        [pltpu.SemaphoreType.DMA, pltpu.SemaphoreType.DMA],
        [pltpu.SemaphoreType.DMA, pltpu.SemaphoreType.DMA],
    ],
  )
  def _sc_gather(x_hbm, y_hbm, tab_hbm, out_hbm, tab_sh,
                 xv, yv, i00, i10, fxv, fyv, g0, g1, ov,
                 sem_in, sem_g, sem_out):
    s_id = lax.axis_index("s")
    wid = s_id * 2 + lax.axis_index("c")
    wrow = wid * _ROWS_W

    seg = (_NB * _NB) // 16
    pltpu.sync_copy(tab_hbm.at[pl.ds(s_id * seg, seg)],
                    tab_sh.at[pl.ds(s_id * seg, seg)])
    plsc.subcore_barrier()

    def fire_in(t, b):
        rb = wrow + t * _RCH
        pltpu.async_copy(x_hbm.at[pl.ds(rb, _RCH), :], xv[b], sem_in[b])
        pltpu.async_copy(y_hbm.at[pl.ds(rb, _RCH), :], yv[b], sem_in[b])

    def wait_in(b):
        pltpu.make_async_copy(
            x_hbm.at[pl.ds(0, _RCH), :], xv[b], sem_in[b]).wait()
        pltpu.make_async_copy(
            y_hbm.at[pl.ds(0, _RCH), :], yv[b], sem_in[b]).wait()

    def compute_idx(b):
        for v in range(_NVS):
            r, j = v // _VPR, v % _VPR
            c = _cstart(j)
            p = v * 16
            sub, off = p // 128, p % 128
            u = xv[b][r, pl.ds(c, 16)] * _INV_DX + (_INPUT_RANGE * _INV_DX)
            w = yv[b][r, pl.ds(c, 16)] * _INV_DX + (_INPUT_RANGE * _INV_DX)
            cx = jnp.clip(u.astype(jnp.int32), 0, _NB - 2)
            cy = jnp.clip(w.astype(jnp.int32), 0, _NB - 2)
            fxv[b][pl.ds(p, 16)] = u - cx.astype(jnp.float32)
            fyv[b][pl.ds(p, 16)] = w - cy.astype(jnp.float32)
            base_idx = cy * _NB + cx
            i00[b][sub, pl.ds(off, 16)] = base_idx
            i10[b][sub, pl.ds(off, 16)] = base_idx + _NB

    def fire_gathers(b):
        for j in range(_NSUB):
            pltpu.async_copy(tab_sh.at[i00[b].at[j]], g0[b].at[j], sem_g[b])
            pltpu.async_copy(tab_sh.at[i10[b].at[j]], g1[b].at[j], sem_g[b])

    def wait_gathers(b):
        for j in range(_NSUB):
            pltpu.make_async_copy(
                tab_hbm.at[pl.ds(0, 128)], g0[b].at[j], sem_g[b]).wait()
            pltpu.make_async_copy(
                tab_hbm.at[pl.ds(0, 128)], g1[b].at[j], sem_g[b]).wait()

    def combine(b):
        for v in range(_NVS):
            r, j = v // _VPR, v % _VPR
            c = _cstart(j)
            p = v * 16
            sub, off = p // 128, p % 128
            w0 = g0[b][sub, pl.ds(off, 16)]
            w1 = g1[b][sub, pl.ds(off, 16)]
            z00 = lax.bitcast_convert_type(w0 << 16, jnp.float32)
            z01 = lax.bitcast_convert_type(w0 & jnp.int32(-65536), jnp.float32)
            z10 = lax.bitcast_convert_type(w1 << 16, jnp.float32)
            z11 = lax.bitcast_convert_type(w1 & jnp.int32(-65536), jnp.float32)
            fx = fxv[b][pl.ds(p, 16)]
            fy = fyv[b][pl.ds(p, 16)]
            top = z00 + fx * (z01 - z00)
            bot = z10 + fx * (z11 - z10)
            ov[b][r, pl.ds(c, 16)] = top + fy * (bot - top)

    def fire_out(t, b):
        rb = wrow + t * _RCH
        pltpu.async_copy(ov[b], out_hbm.at[pl.ds(rb, _RCH), :], sem_out[b])

    def wait_out(b):
        pltpu.make_async_copy(
            x_hbm.at[pl.ds(0, _RCH), :], ov[b], sem_out[b]).wait()

    fire_in(0, 0)
    fire_in(1, 1)

    def body(k, carry):
        t = 2 * k
        wait_in(0)
        compute_idx(0)

        @pl.when(t + 2 < _NCHUNK)
        def _():
            fire_in(t + 2, 0)
        fire_gathers(0)

        @pl.when(k > 0)
        def _():
            wait_gathers(1)

            @pl.when(k > 1)
            def _():
                wait_out(1)
            combine(1)
            fire_out(t - 1, 1)

        wait_in(1)
        compute_idx(1)

        @pl.when(t + 3 < _NCHUNK)
        def _():
            fire_in(t + 3, 1)
        fire_gathers(1)

        wait_gathers(0)

        @pl.when(k > 0)
        def _():
            wait_out(0)
        combine(0)
        fire_out(t, 0)
        return carry

    lax.fori_loop(0, _NCHUNK // 2, body, 0)

    wait_gathers(1)
    wait_out(1)
    combine(1)
    fire_out(_NCHUNK - 1, 1)
    wait_out(0)
    wait_out(1)

  return _sc_gather


def kernel(x, y, inv_softplus_step_values):
    words = _table_tc(inv_softplus_step_values).reshape(-1)
    return _make_sc_gather()(x, y, words)


# speedup vs baseline: 4.9794x; 1.1649x over previous
import functools

import jax
import jax.numpy as jnp
from jax import lax
from jax.experimental import pallas as pl
from jax.experimental.pallas import tpu as pltpu
from jax.experimental.pallas import tpu_sc as plsc

_INPUT_RANGE = 4.0
_NB = 1024
_DX = 2.0 * _INPUT_RANGE / (_NB - 1)
_INV_DX = 1.0 / _DX

_ROWS, _COLS = 16384, 100
_NW = 32
_ROWS_W = _ROWS // _NW
_RCH = 4
_NCHUNK = _ROWS_W // _RCH
_VPR = 7
_NVS = _RCH * _VPR
_SUBW = _VPR * 16


def _cint_body(w_ref, out_ref):
    w = w_ref[...]
    sp = jnp.maximum(w, 0.0) + jnp.log1p(jnp.exp(-jnp.abs(w)))
    cell = sp * (_DX * _DX)
    row_i = lax.broadcasted_iota(jnp.int32, (_NB, _NB), 0)
    col_i = lax.broadcasted_iota(jnp.int32, (_NB, _NB), 1)
    tri = (row_i <= col_i).astype(jnp.float32)
    cs1 = lax.dot(cell, tri)
    cs2 = lax.dot_general(tri, cs1, (((0,), (0,)), ((), ())))
    a = cs2[0:1, 0:1]
    b = cs2[_NB - 1:_NB, _NB - 1:_NB]
    scale = (2.0 * _INPUT_RANGE) / (b - a)
    cint = (cs2 - a) * scale - _INPUT_RANGE
    cint_r = jnp.concatenate([cint[:, 1:], cint[:, _NB - 1:_NB]], axis=1)
    lo = lax.bitcast_convert_type(cint.astype(jnp.bfloat16),
                                  jnp.uint16).astype(jnp.uint32)
    hi = lax.bitcast_convert_type(cint_r.astype(jnp.bfloat16),
                                  jnp.uint16).astype(jnp.uint32)
    out_ref[...] = (lo | (hi << 16)).astype(jnp.int32)


def _table_tc(w):
    return pl.pallas_call(
        _cint_body,
        out_shape=jax.ShapeDtypeStruct((_NB, _NB), jnp.int32),
    )(w)


def _cstart(j):
    return 16 * j if j < _VPR - 1 else _COLS - 16


@functools.lru_cache(maxsize=1)
def _make_sc_gather():
  mesh = plsc.VectorSubcoreMesh(core_axis_name="c", subcore_axis_name="s")

  io2d = lambda: pltpu.VMEM((_RCH, _COLS), jnp.float32)
  flatbuf = lambda: pltpu.VMEM((_NVS * 16,), jnp.float32)
  idxbuf = lambda: pltpu.VMEM((_RCH, _SUBW), jnp.int32)
  gbuf = lambda: pltpu.VMEM((_RCH, _SUBW), jnp.int32)

  @functools.partial(
    pl.kernel,
    mesh=mesh,
    out_type=jax.ShapeDtypeStruct((_ROWS, _COLS), jnp.float32),
    scratch_types=[
        pltpu.VMEM_SHARED((_NB * _NB,), jnp.int32),
        [io2d(), io2d()],
        [io2d(), io2d()],
        [idxbuf(), idxbuf()],
        [idxbuf(), idxbuf()],
        [flatbuf(), flatbuf()],
        [flatbuf(), flatbuf()],
        [gbuf(), gbuf()],
        [gbuf(), gbuf()],
        [io2d(), io2d()],
        [pltpu.SemaphoreType.DMA, pltpu.SemaphoreType.DMA],
        [pltpu.SemaphoreType.DMA, pltpu.SemaphoreType.DMA],
        [pltpu.SemaphoreType.DMA, pltpu.SemaphoreType.DMA],
    ],
  )
  def _sc_gather(x_hbm, y_hbm, tab_hbm, out_hbm, tab_sh,
                 xv, yv, i00, i10, fxv, fyv, g0, g1, ov,
                 sem_in, sem_g, sem_out):
    s_id = lax.axis_index("s")
    wid = s_id * 2 + lax.axis_index("c")
    wrow = wid * _ROWS_W

    seg = (_NB * _NB) // 16
    pltpu.sync_copy(tab_hbm.at[pl.ds(s_id * seg, seg)],
                    tab_sh.at[pl.ds(s_id * seg, seg)])
    plsc.subcore_barrier()

    def fire_in(t, b):
        rb = wrow + t * _RCH
        pltpu.async_copy(x_hbm.at[pl.ds(rb, _RCH), :], xv[b], sem_in[b])
        pltpu.async_copy(y_hbm.at[pl.ds(rb, _RCH), :], yv[b], sem_in[b])

    def wait_in(b):
        pltpu.make_async_copy(
            x_hbm.at[pl.ds(0, _RCH), :], xv[b], sem_in[b]).wait()
        pltpu.make_async_copy(
            y_hbm.at[pl.ds(0, _RCH), :], yv[b], sem_in[b]).wait()

    def compute_idx(b):
        for v in range(_NVS):
            r, j = v // _VPR, v % _VPR
            c = _cstart(j)
            sub, off = r, 16 * j
            p = v * 16
            u = xv[b][r, pl.ds(c, 16)] * _INV_DX + (_INPUT_RANGE * _INV_DX)
            w = yv[b][r, pl.ds(c, 16)] * _INV_DX + (_INPUT_RANGE * _INV_DX)
            cx = jnp.clip(u.astype(jnp.int32), 0, _NB - 2)
            cy = jnp.clip(w.astype(jnp.int32), 0, _NB - 2)
            fxv[b][pl.ds(p, 16)] = u - cx.astype(jnp.float32)
            fyv[b][pl.ds(p, 16)] = w - cy.astype(jnp.float32)
            base_idx = cy * _NB + cx
            i00[b][sub, pl.ds(off, 16)] = base_idx
            i10[b][sub, pl.ds(off, 16)] = base_idx + _NB

    def fire_gathers(b):
        for j in range(_RCH):
            pltpu.async_copy(tab_sh.at[i00[b].at[j]], g0[b].at[j], sem_g[b])
            pltpu.async_copy(tab_sh.at[i10[b].at[j]], g1[b].at[j], sem_g[b])

    def wait_gathers(b):
        for j in range(_RCH):
            pltpu.make_async_copy(
                tab_hbm.at[pl.ds(0, _SUBW)], g0[b].at[j], sem_g[b]).wait()
            pltpu.make_async_copy(
                tab_hbm.at[pl.ds(0, _SUBW)], g1[b].at[j], sem_g[b]).wait()

    def combine(b):
        for v in range(_NVS):
            r, j = v // _VPR, v % _VPR
            c = _cstart(j)
            sub, off = r, 16 * j
            p = v * 16
            w0 = g0[b][sub, pl.ds(off, 16)]
            w1 = g1[b][sub, pl.ds(off, 16)]
            z00 = lax.bitcast_convert_type(w0 << 16, jnp.float32)
            z01 = lax.bitcast_convert_type(w0 & jnp.int32(-65536), jnp.float32)
            z10 = lax.bitcast_convert_type(w1 << 16, jnp.float32)
            z11 = lax.bitcast_convert_type(w1 & jnp.int32(-65536), jnp.float32)
            fx = fxv[b][pl.ds(p, 16)]
            fy = fyv[b][pl.ds(p, 16)]
            top = z00 + fx * (z01 - z00)
            bot = z10 + fx * (z11 - z10)
            ov[b][r, pl.ds(c, 16)] = top + fy * (bot - top)

    def fire_out(t, b):
        rb = wrow + t * _RCH
        pltpu.async_copy(ov[b], out_hbm.at[pl.ds(rb, _RCH), :], sem_out[b])

    def wait_out(b):
        pltpu.make_async_copy(
            x_hbm.at[pl.ds(0, _RCH), :], ov[b], sem_out[b]).wait()

    fire_in(0, 0)
    fire_in(1, 1)

    def body(k, carry):
        t = 2 * k
        wait_in(0)
        compute_idx(0)

        @pl.when(t + 2 < _NCHUNK)
        def _():
            fire_in(t + 2, 0)
        fire_gathers(0)

        @pl.when(k > 0)
        def _():
            wait_gathers(1)

            @pl.when(k > 1)
            def _():
                wait_out(1)
            combine(1)
            fire_out(t - 1, 1)

        wait_in(1)
        compute_idx(1)

        @pl.when(t + 3 < _NCHUNK)
        def _():
            fire_in(t + 3, 1)
        fire_gathers(1)

        wait_gathers(0)

        @pl.when(k > 0)
        def _():
            wait_out(0)
        combine(0)
        fire_out(t, 0)
        return carry

    lax.fori_loop(0, _NCHUNK // 2, body, 0)

    wait_gathers(1)
    wait_out(1)
    combine(1)
    fire_out(_NCHUNK - 1, 1)
    wait_out(0)
    wait_out(1)

  return _sc_gather


def kernel(x, y, inv_softplus_step_values):
    words = _table_tc(inv_softplus_step_values).reshape(-1)
    return _make_sc_gather()(x, y, words)
